# baseline (speedup 1.0000x reference)
import functools

import jax
import jax.numpy as jnp
from jax import lax
from jax.experimental import pallas as pl
from jax.experimental.pallas import tpu as pltpu
from jax.experimental.pallas import tpu_sc as plsc

NC = 2
NS = 16
L = 16
NW = NC * NS

B = 1024
S = 16
D = 128
E = 128

B_PER_W = B // NW
ROWS_PER_W = B_PER_W * S
GW = 128
N_CH = ROWS_PER_W // GW

_MESH = plsc.VectorSubcoreMesh(core_axis_name="c", subcore_axis_name="s")


def _sc_gather_self(features, nodes):
    @functools.partial(
        pl.kernel,
        out_type=jax.ShapeDtypeStruct((B, D), jnp.float32),
        mesh=_MESH,
        scratch_types=[
            pltpu.VMEM((B_PER_W,), jnp.int32),
            pltpu.VMEM((B_PER_W, D), jnp.float32),
            pltpu.SemaphoreType.DMA,
        ],
    )
    def k(feat_hbm, nodes_hbm, self_out, sidx_v, self_v, sem):
        wid = lax.axis_index("s") * NC + lax.axis_index("c")
        base = wid * B_PER_W
        pltpu.sync_copy(nodes_hbm.at[pl.ds(base, B_PER_W)], sidx_v)
        pltpu.async_copy(feat_hbm.at[sidx_v], self_v, sem).wait()
        pltpu.sync_copy(self_v, self_out.at[pl.ds(base, B_PER_W)])

    return k(features, nodes)


def _sc_gather_mean(features, neigh_flat):
    @functools.partial(
        pl.kernel,
        out_type=jax.ShapeDtypeStruct((B, D), jnp.float32),
        mesh=_MESH,
        scratch_types=[
            pltpu.VMEM((ROWS_PER_W,), jnp.int32),
            pltpu.VMEM((ROWS_PER_W, D), jnp.float32),
            pltpu.VMEM((B_PER_W, D), jnp.float32),
            pltpu.SemaphoreType.DMA,
        ],
    )
    def k(feat_hbm, nidx_hbm, mean_out, nidx_v, rows_v, mean_v, sem):
        wid = lax.axis_index("s") * NC + lax.axis_index("c")
        base = wid * B_PER_W
        rbase = wid * ROWS_PER_W

        pltpu.sync_copy(nidx_hbm.at[pl.ds(rbase, ROWS_PER_W)], nidx_v)

        copies = []
        for j in range(N_CH):
            copies.append(pltpu.async_copy(
                feat_hbm.at[nidx_v.at[pl.ds(j * GW, GW)]],
                rows_v.at[pl.ds(j * GW, GW)], sem))
        for c in copies:
            c.wait()

        @pl.loop(0, B_PER_W)
        def _(n):
            row0 = n * S
            for c in range(D // L):
                cs = pl.ds(c * L, L)
                acc = rows_v[row0, cs]
                for r in range(1, S):
                    acc = acc + rows_v[row0 + r, cs]
                mean_v[n, cs] = acc * (1.0 / S)

        pltpu.sync_copy(mean_v, mean_out.at[pl.ds(base, B_PER_W)])

    return k(features, neigh_flat)


def _tc_bmm_half(x, weight, half, partial=None):
    Bb = 64

    def body_first(x_ref, w_ref, o_ref):
        acc = jax.lax.dot_general(
            x_ref[...], w_ref[:, 0, :, :],
            dimension_numbers=(((1,), (1,)), ((0,), (0,))),
            preferred_element_type=jnp.float32)
        o_ref[...] = acc

    def body_second(x_ref, w_ref, p_ref, o_ref):
        acc = jax.lax.dot_general(
            x_ref[...], w_ref[:, 0, :, :],
            dimension_numbers=(((1,), (1,)), ((0,), (0,))),
            preferred_element_type=jnp.float32)
        o_ref[...] = jnp.maximum(p_ref[...] + acc, 0.0)

    w4 = weight.reshape(B, 2, D, E)
    in_specs = [
        pl.BlockSpec((Bb, D), lambda i: (i, 0)),
        pl.BlockSpec((Bb, 1, D, E), lambda i: (i, half, 0, 0)),
    ]
    args = [x, w4]
    body = body_first
    if partial is not None:
        in_specs.append(pl.BlockSpec((Bb, E), lambda i: (i, 0)))
        args.append(partial)
        body = body_second

    return pl.pallas_call(
        body,
        grid=(B // Bb,),
        in_specs=in_specs,
        out_specs=pl.BlockSpec((Bb, E), lambda i: (i, 0)),
        out_shape=jax.ShapeDtypeStruct((B, E), jnp.float32),
    )(*args)


def kernel(features, nodes, neigh_idx, weight):
    nodes = nodes.astype(jnp.int32)
    neigh_flat = neigh_idx.astype(jnp.int32).reshape(-1)
    selff = _sc_gather_self(features, nodes)
    meanf = _sc_gather_mean(features, neigh_flat)
    sp = _tc_bmm_half(selff, weight, 0)
    return _tc_bmm_half(meanf, weight, 1, partial=sp)


# speedup vs baseline: 1.0105x; 1.0105x over previous
import functools

import jax
import jax.numpy as jnp
from jax import lax
from jax.experimental import pallas as pl
from jax.experimental.pallas import tpu as pltpu
from jax.experimental.pallas import tpu_sc as plsc

NC = 2
NS = 16
L = 16
NW = NC * NS

B = 1024
S = 16
D = 128
E = 128

B_PER_W = B // NW
ROWS_PER_W = B_PER_W * S
GW = 128
N_CH = ROWS_PER_W // GW
NODES_PER_CH = GW // S

_MESH = plsc.VectorSubcoreMesh(core_axis_name="c", subcore_axis_name="s")


def _sc_gather_mean(features, nodes, neigh_flat):
    @functools.partial(
        pl.kernel,
        out_type=(
            jax.ShapeDtypeStruct((B, D), jnp.float32),
            jax.ShapeDtypeStruct((B, D), jnp.float32),
        ),
        mesh=_MESH,
        scratch_types=[
            pltpu.VMEM((ROWS_PER_W,), jnp.int32),
            pltpu.VMEM((B_PER_W,), jnp.int32),
            pltpu.VMEM((ROWS_PER_W, D), jnp.float32),
            pltpu.VMEM((B_PER_W, D), jnp.float32),
            pltpu.VMEM((B_PER_W, D), jnp.float32),
            pltpu.SemaphoreType.DMA,
            pltpu.SemaphoreType.DMA,
        ],
    )
    def k(feat_hbm, nodes_hbm, nidx_hbm, self_out, mean_out,
          nidx_v, sidx_v, rows_v, self_v, mean_v, sem, sem2):
        wid = lax.axis_index("s") * NC + lax.axis_index("c")
        base = wid * B_PER_W
        rbase = wid * ROWS_PER_W

        pltpu.sync_copy(nidx_hbm.at[pl.ds(rbase, ROWS_PER_W)], nidx_v)
        pltpu.sync_copy(nodes_hbm.at[pl.ds(base, B_PER_W)], sidx_v)

        copies = []
        for j in range(N_CH):
            copies.append(pltpu.async_copy(
                feat_hbm.at[nidx_v.at[pl.ds(j * GW, GW)]],
                rows_v.at[pl.ds(j * GW, GW)], sem))
        self_copy = pltpu.async_copy(feat_hbm.at[sidx_v], self_v, sem2)

        for j in range(N_CH):
            copies[j].wait()

            @pl.loop(j * NODES_PER_CH, (j + 1) * NODES_PER_CH)
            def _(n):
                row0 = n * S
                for c in range(D // L):
                    cs = pl.ds(c * L, L)
                    acc = rows_v[row0, cs]
                    for r in range(1, S):
                        acc = acc + rows_v[row0 + r, cs]
                    mean_v[n, cs] = acc * (1.0 / S)

        self_copy.wait()
        pltpu.sync_copy(self_v, self_out.at[pl.ds(base, B_PER_W)])
        pltpu.sync_copy(mean_v, mean_out.at[pl.ds(base, B_PER_W)])

    return k(features, nodes, neigh_flat)


def _tc_bmm(selff, meanf, weight):
    Bb = 64

    def body(s_ref, m_ref, w_ref, o_ref):
        c = jnp.concatenate([s_ref[...], m_ref[...]], axis=1)
        acc = jax.lax.dot_general(
            c, w_ref[...],
            dimension_numbers=(((1,), (1,)), ((0,), (0,))),
            preferred_element_type=jnp.float32)
        o_ref[...] = jnp.maximum(acc, 0.0)

    return pl.pallas_call(
        body,
        grid=(B // Bb,),
        in_specs=[
            pl.BlockSpec((Bb, D), lambda i: (i, 0)),
            pl.BlockSpec((Bb, D), lambda i: (i, 0)),
            pl.BlockSpec((Bb, 2 * D, E), lambda i: (i, 0, 0)),
        ],
        out_specs=pl.BlockSpec((Bb, E), lambda i: (i, 0)),
        out_shape=jax.ShapeDtypeStruct((B, E), jnp.float32),
    )(selff, meanf, weight)


def kernel(features, nodes, neigh_idx, weight):
    nodes = nodes.astype(jnp.int32)
    neigh_flat = neigh_idx.astype(jnp.int32).reshape(-1)
    selff, meanf = _sc_gather_mean(features, nodes, neigh_flat)
    return _tc_bmm(selff, meanf, weight)


# speedup vs baseline: 1.0684x; 1.0573x over previous
import functools

import jax
import jax.numpy as jnp
from jax import lax
from jax.experimental import pallas as pl
from jax.experimental.pallas import tpu as pltpu
from jax.experimental.pallas import tpu_sc as plsc

NC = 2
NS = 16
L = 16
NW = NC * NS

B = 1024
S = 16
D = 128
E = 128

B_PER_W = B // NW
ROWS_PER_W = B_PER_W * S
GW = 128
N_CH = ROWS_PER_W // GW
NODES_PER_CH = GW // S

_MESH = plsc.VectorSubcoreMesh(core_axis_name="c", subcore_axis_name="s")


def _sc_gather_mean(features, nodes, neigh_flat):
    @functools.partial(
        pl.kernel,
        out_type=(
            jax.ShapeDtypeStruct((B, D), jnp.float32),
            jax.ShapeDtypeStruct((B, D), jnp.float32),
        ),
        mesh=_MESH,
        scratch_types=[
            pltpu.VMEM((ROWS_PER_W,), jnp.int32),
            pltpu.VMEM((B_PER_W,), jnp.int32),
            pltpu.VMEM((ROWS_PER_W, D), jnp.float32),
            pltpu.VMEM((B_PER_W, D), jnp.float32),
            pltpu.VMEM((B_PER_W, D), jnp.float32),
            pltpu.VMEM_SHARED((NS * B_PER_W, D), jnp.float32),
            [pltpu.SemaphoreType.DMA] * N_CH,
            pltpu.SemaphoreType.DMA,
            pltpu.SemaphoreType.DMA,
        ],
    )
    def k(feat_hbm, nodes_hbm, nidx_hbm, self_out, mean_out,
          nidx_v, sidx_v, rows_v, self_v, mean_v, acc_sh,
          gsems, ssem, asem):
        sid = lax.axis_index("s")
        cid = lax.axis_index("c")
        wid = sid * NC + cid
        base = wid * B_PER_W
        sbase = sid * B_PER_W

        pltpu.sync_copy(nidx_hbm.at[pl.ds(wid * ROWS_PER_W, ROWS_PER_W)],
                        nidx_v)
        pltpu.sync_copy(nodes_hbm.at[pl.ds(base, B_PER_W)], sidx_v)

        gcopies = []
        for j in range(N_CH):
            gcopies.append(pltpu.async_copy(
                feat_hbm.at[nidx_v.at[pl.ds(j * GW, GW)]],
                rows_v.at[pl.ds(j * GW, GW)], gsems[j]))
        self_copy = pltpu.async_copy(feat_hbm.at[sidx_v], self_v, ssem)

        @pl.loop(0, B_PER_W)
        def _(n):
            for c in range(D // L):
                mean_v[n, pl.ds(c * L, L)] = jnp.zeros((L,), jnp.float32)

        pltpu.sync_copy(mean_v, acc_sh.at[pl.ds(sbase, B_PER_W)])

        acopies = []
        for j in range(N_CH):
            gcopies[j].wait()
            for n in range(j * NODES_PER_CH, (j + 1) * NODES_PER_CH):
                dst_rows = (sbase + n) + jnp.zeros((L,), jnp.int32)
                acopies.append(pltpu.async_copy(
                    rows_v.at[pl.ds(n * S, S)],
                    acc_sh.at[dst_rows], asem, add=True))
        for c in acopies:
            c.wait()

        pltpu.sync_copy(acc_sh.at[pl.ds(sbase, B_PER_W)], mean_v)

        @pl.loop(0, B_PER_W)
        def _(n):
            for c in range(D // L):
                cs = pl.ds(c * L, L)
                mean_v[n, cs] = mean_v[n, cs] * (1.0 / S)

        self_copy.wait()
        pltpu.sync_copy(self_v, self_out.at[pl.ds(base, B_PER_W)])
        pltpu.sync_copy(mean_v, mean_out.at[pl.ds(base, B_PER_W)])

    return k(features, nodes, neigh_flat)


def _tc_bmm(selff, meanf, weight):
    Bb = 64

    def body(s_ref, m_ref, w_ref, o_ref):
        c = jnp.concatenate([s_ref[...], m_ref[...]], axis=1)
        acc = jax.lax.dot_general(
            c, w_ref[...],
            dimension_numbers=(((1,), (1,)), ((0,), (0,))),
            preferred_element_type=jnp.float32)
        o_ref[...] = jnp.maximum(acc, 0.0)

    return pl.pallas_call(
        body,
        grid=(B // Bb,),
        in_specs=[
            pl.BlockSpec((Bb, D), lambda i: (i, 0)),
            pl.BlockSpec((Bb, D), lambda i: (i, 0)),
            pl.BlockSpec((Bb, 2 * D, E), lambda i: (i, 0, 0)),
        ],
        out_specs=pl.BlockSpec((Bb, E), lambda i: (i, 0)),
        out_shape=jax.ShapeDtypeStruct((B, E), jnp.float32),
    )(selff, meanf, weight)


def kernel(features, nodes, neigh_idx, weight):
    nodes = nodes.astype(jnp.int32)
    neigh_flat = neigh_idx.astype(jnp.int32).reshape(-1)
    selff, meanf = _sc_gather_mean(features, nodes, neigh_flat)
    return _tc_bmm(selff, meanf, weight)
